# baseline (speedup 1.0000x reference)
import functools

import jax
import jax.numpy as jnp
from jax import lax
from jax.experimental import pallas as pl
from jax.experimental.pallas import tpu as pltpu
from jax.experimental.pallas import tpu_sc as plsc

_V, _D = 1_000_000, 64
_B, _C, _K = 16384, 20, 20
_NC, _NS = 2, 16
_NW = _NC * _NS
_BPW = _B // _NW
_NB = 16
_NCHUNK = _BPW // _NB
_ROWS = _NB * _C
_NSTREAM = 4
_SROWS = _ROWS // _NSTREAM
_L = 16


def _sc_body(idxu_hbm, idxn_hbm, idxt_hbm, emb_u_hbm, emb_v_hbm,
             pos_hbm, neg_hbm,
             idx_u, idx_n, idx_t, rows_u, rows_n, rows_t,
             pos_buf, neg_buf, sem):
    wid = lax.axis_index("s") * _NC + lax.axis_index("c")
    base = wid * _BPW
    pltpu.sync_copy(idxu_hbm.at[pl.ds(base * _C, _BPW * _C)], idx_u)
    pltpu.sync_copy(idxn_hbm.at[pl.ds(base * _K, _BPW * _K)], idx_n)
    pltpu.sync_copy(idxt_hbm.at[pl.ds(base, _BPW)], idx_t)

    lanes = lax.iota(jnp.int32, _L)

    def _lane_sum(x):
        for s in (8, 4, 2, 1):
            x = x + x.at[(lanes + s) % _L].get(mode="promise_in_bounds")
        return x

    def chunk_body(ci, _):
        off = ci * _ROWS
        cps = []
        for s in range(_NSTREAM):
            cps.append(pltpu.async_copy(
                emb_u_hbm.at[idx_u.at[pl.ds(off + s * _SROWS, _SROWS)]],
                rows_u.at[pl.ds(s * _SROWS, _SROWS), :], sem))
            cps.append(pltpu.async_copy(
                emb_v_hbm.at[idx_n.at[pl.ds(off + s * _SROWS, _SROWS)]],
                rows_n.at[pl.ds(s * _SROWS, _SROWS), :], sem))
        cps.append(pltpu.async_copy(
            emb_v_hbm.at[idx_t.at[pl.ds(ci * _NB, _NB)]], rows_t, sem))
        for cp in cps:
            cp.wait()

        def b_body(j, carry):
            acc_p, acc_n = carry
            rb = j * _C
            su = [jnp.zeros((_L,), jnp.float32) for _ in range(_D // _L)]
            sn = [jnp.zeros((_L,), jnp.float32) for _ in range(_D // _L)]
            for c in range(_C):
                for blk in range(_D // _L):
                    su[blk] = su[blk] + rows_u[rb + c, pl.ds(blk * _L, _L)]
                    sn[blk] = sn[blk] + rows_n[rb + c, pl.ds(blk * _L, _L)]
            pp = jnp.zeros((_L,), jnp.float32)
            nn = jnp.zeros((_L,), jnp.float32)
            for blk in range(_D // _L):
                t_blk = rows_t[j, pl.ds(blk * _L, _L)]
                pp = pp + t_blk * su[blk]
                nn = nn + sn[blk] * su[blk]
            m = lanes == j
            acc_p = jnp.where(m, _lane_sum(pp) * (1.0 / _C), acc_p)
            acc_n = jnp.where(m, _lane_sum(nn) * (1.0 / _C), acc_n)
            return acc_p, acc_n

        acc_p, acc_n = lax.fori_loop(
            0, _NB, b_body,
            (jnp.zeros((_L,), jnp.float32), jnp.zeros((_L,), jnp.float32)))
        pos_buf[pl.ds(ci * _NB, _NB)] = acc_p
        neg_buf[pl.ds(ci * _NB, _NB)] = acc_n
        return 0

    lax.fori_loop(0, _NCHUNK, chunk_body, 0)
    pltpu.sync_copy(pos_buf, pos_hbm.at[pl.ds(base, _BPW)])
    pltpu.sync_copy(neg_buf, neg_hbm.at[pl.ds(base, _BPW)])


@functools.lru_cache(maxsize=None)
def _sc_scores():
    return functools.partial(
        pl.kernel,
        mesh=plsc.VectorSubcoreMesh(core_axis_name="c", subcore_axis_name="s"),
        compiler_params=pltpu.CompilerParams(use_tc_tiling_on_sc=False),
        out_type=[jax.ShapeDtypeStruct((_B,), jnp.float32),
                  jax.ShapeDtypeStruct((_B,), jnp.float32)],
        scratch_types=[
            pltpu.VMEM((_BPW * _C,), jnp.int32),
            pltpu.VMEM((_BPW * _K,), jnp.int32),
            pltpu.VMEM((_BPW,), jnp.int32),
            pltpu.VMEM((_ROWS, _D), jnp.float32),
            pltpu.VMEM((_ROWS, _D), jnp.float32),
            pltpu.VMEM((_NB, _D), jnp.float32),
            pltpu.VMEM((_BPW,), jnp.float32),
            pltpu.VMEM((_BPW,), jnp.float32),
            pltpu.SemaphoreType.DMA,
        ],
    )(_sc_body)


def _finish_body(pos_ref, neg_ref, out_ref):
    p = pos_ref[...]
    n = neg_ref[...]
    lsp = jnp.minimum(p, 0.0) - jnp.log1p(jnp.exp(-jnp.abs(p)))
    lsn = jnp.minimum(-n, 0.0) - jnp.log1p(jnp.exp(-jnp.abs(n)))
    out_ref[0, 0] = -(jnp.sum(lsp) + jnp.sum(lsn)) / _B


def _finish(pos2d, neg2d):
    return pl.pallas_call(
        _finish_body,
        out_shape=jax.ShapeDtypeStruct((1, 1), jnp.float32),
        out_specs=pl.BlockSpec(memory_space=pltpu.SMEM),
    )(pos2d, neg2d)


_RBLK = 400
_H = _V // 2


def _repack_body(in1_ref, in2_ref, out_ref):
    out_ref[:, 0:_D] = in1_ref[...]
    out_ref[:, _D:2 * _D] = in2_ref[...]


def _repack(table):
    out = pl.pallas_call(
        _repack_body,
        grid=(_H // _RBLK,),
        in_specs=[pl.BlockSpec((_RBLK, _D), lambda i: (i, 0)),
                  pl.BlockSpec((_RBLK, _D), lambda i: (i + _H // _RBLK, 0))],
        out_specs=pl.BlockSpec((_RBLK, 2 * _D), lambda i: (i, 0)),
        out_shape=jax.ShapeDtypeStruct((_H, 2 * _D), jnp.float32),
    )(table, table)
    return out.reshape(_V, _D)


def _remap(v):
    h = (v >= _H).astype(jnp.int32)
    return 2 * (v - h * _H) + h


def kernel(inputs, targets, negatives, emb_u, emb_v):
    idxu = _remap(inputs.astype(jnp.int32)).reshape(-1)
    idxn = _remap(negatives.astype(jnp.int32)).reshape(-1)
    idxt = _remap(targets.astype(jnp.int32))
    pos, neg = _sc_scores()(idxu, idxn, idxt, _repack(emb_u), _repack(emb_v))
    res = _finish(pos.reshape(128, 128), neg.reshape(128, 128))
    return res[0, 0]


# speedup vs baseline: 2.0689x; 2.0689x over previous
import functools

import jax
import jax.numpy as jnp
from jax import lax
from jax.experimental import pallas as pl
from jax.experimental.pallas import tpu as pltpu
from jax.experimental.pallas import tpu_sc as plsc

_V, _D = 1_000_000, 64
_B, _C, _K = 16384, 20, 20
_NC, _NS = 2, 16
_NW = _NC * _NS
_BPW = _B // _NW
_NB = 16
_NCHUNK = _BPW // _NB
_ROWS = _NB * _C
_NSTREAM = 4
_SROWS = _ROWS // _NSTREAM
_L = 16


def _phase_a_body(idxu_hbm, emb_u_hbm, su_hbm,
                  idx_u, rows_u, su_buf, sem):
    wid = lax.axis_index("s") * _NC + lax.axis_index("c")
    base = wid * _BPW
    pltpu.sync_copy(idxu_hbm.at[pl.ds(base * _C, _BPW * _C)], idx_u)

    def chunk_body(ci, _):
        off = ci * _ROWS
        cps = [pltpu.async_copy(
            emb_u_hbm.at[idx_u.at[pl.ds(off + s * _SROWS, _SROWS)]],
            rows_u.at[pl.ds(s * _SROWS, _SROWS), :], sem)
            for s in range(_NSTREAM)]
        for cp in cps:
            cp.wait()

        def b_body(j, _):
            rb = j * _C
            su = [jnp.zeros((_L,), jnp.float32) for _ in range(_D // _L)]
            for c in range(_C):
                for blk in range(_D // _L):
                    su[blk] = su[blk] + rows_u[rb + c, pl.ds(blk * _L, _L)]
            for blk in range(_D // _L):
                su_buf[ci * _NB + j, pl.ds(blk * _L, _L)] = su[blk]
            return 0

        lax.fori_loop(0, _NB, b_body, 0)
        return 0

    lax.fori_loop(0, _NCHUNK, chunk_body, 0)
    pltpu.sync_copy(su_buf, su_hbm.at[pl.ds(base, _BPW), :])


def _phase_b_body(idxn_hbm, idxt_hbm, su_hbm, emb_v_hbm,
                  pos_hbm, neg_hbm,
                  idx_n, idx_t, rows_n, rows_t, su_buf,
                  pos_buf, neg_buf, sem):
    wid = lax.axis_index("s") * _NC + lax.axis_index("c")
    base = wid * _BPW
    pltpu.sync_copy(idxn_hbm.at[pl.ds(base * _K, _BPW * _K)], idx_n)
    pltpu.sync_copy(idxt_hbm.at[pl.ds(base, _BPW)], idx_t)
    pltpu.sync_copy(su_hbm.at[pl.ds(base, _BPW), :], su_buf)

    lanes = lax.iota(jnp.int32, _L)

    def _lane_sum(x):
        for s in (8, 4, 2, 1):
            x = x + x.at[(lanes + s) % _L].get(mode="promise_in_bounds")
        return x

    def chunk_body(ci, _):
        off = ci * _ROWS
        cps = [pltpu.async_copy(
            emb_v_hbm.at[idx_n.at[pl.ds(off + s * _SROWS, _SROWS)]],
            rows_n.at[pl.ds(s * _SROWS, _SROWS), :], sem)
            for s in range(_NSTREAM)]
        cps.append(pltpu.async_copy(
            emb_v_hbm.at[idx_t.at[pl.ds(ci * _NB, _NB)]], rows_t, sem))
        for cp in cps:
            cp.wait()

        def b_body(j, carry):
            acc_p, acc_n = carry
            rb = j * _C
            sn = [jnp.zeros((_L,), jnp.float32) for _ in range(_D // _L)]
            for c in range(_C):
                for blk in range(_D // _L):
                    sn[blk] = sn[blk] + rows_n[rb + c, pl.ds(blk * _L, _L)]
            pp = jnp.zeros((_L,), jnp.float32)
            nn = jnp.zeros((_L,), jnp.float32)
            for blk in range(_D // _L):
                su_blk = su_buf[ci * _NB + j, pl.ds(blk * _L, _L)]
                t_blk = rows_t[j, pl.ds(blk * _L, _L)]
                pp = pp + t_blk * su_blk
                nn = nn + sn[blk] * su_blk
            m = lanes == j
            acc_p = jnp.where(m, _lane_sum(pp) * (1.0 / _C), acc_p)
            acc_n = jnp.where(m, _lane_sum(nn) * (1.0 / _C), acc_n)
            return acc_p, acc_n

        acc_p, acc_n = lax.fori_loop(
            0, _NB, b_body,
            (jnp.zeros((_L,), jnp.float32), jnp.zeros((_L,), jnp.float32)))
        pos_buf[pl.ds(ci * _NB, _NB)] = acc_p
        neg_buf[pl.ds(ci * _NB, _NB)] = acc_n
        return 0

    lax.fori_loop(0, _NCHUNK, chunk_body, 0)
    pltpu.sync_copy(pos_buf, pos_hbm.at[pl.ds(base, _BPW)])
    pltpu.sync_copy(neg_buf, neg_hbm.at[pl.ds(base, _BPW)])


_MESH = dict(core_axis_name="c", subcore_axis_name="s")


@functools.lru_cache(maxsize=None)
def _phase_a():
    return functools.partial(
        pl.kernel,
        mesh=plsc.VectorSubcoreMesh(**_MESH),
        compiler_params=pltpu.CompilerParams(use_tc_tiling_on_sc=False),
        out_type=jax.ShapeDtypeStruct((_B, _D), jnp.float32),
        scratch_types=[
            pltpu.VMEM((_BPW * _C,), jnp.int32),
            pltpu.VMEM((_ROWS, _D), jnp.float32),
            pltpu.VMEM((_BPW, _D), jnp.float32),
            pltpu.SemaphoreType.DMA,
        ],
    )(_phase_a_body)


@functools.lru_cache(maxsize=None)
def _phase_b():
    return functools.partial(
        pl.kernel,
        mesh=plsc.VectorSubcoreMesh(**_MESH),
        compiler_params=pltpu.CompilerParams(use_tc_tiling_on_sc=False),
        out_type=[jax.ShapeDtypeStruct((_B,), jnp.float32),
                  jax.ShapeDtypeStruct((_B,), jnp.float32)],
        scratch_types=[
            pltpu.VMEM((_BPW * _K,), jnp.int32),
            pltpu.VMEM((_BPW,), jnp.int32),
            pltpu.VMEM((_ROWS, _D), jnp.float32),
            pltpu.VMEM((_NB, _D), jnp.float32),
            pltpu.VMEM((_BPW, _D), jnp.float32),
            pltpu.VMEM((_BPW,), jnp.float32),
            pltpu.VMEM((_BPW,), jnp.float32),
            pltpu.SemaphoreType.DMA,
        ],
    )(_phase_b_body)


def _finish_body(pos_ref, neg_ref, out_ref):
    p = pos_ref[...]
    n = neg_ref[...]
    lsp = jnp.minimum(p, 0.0) - jnp.log1p(jnp.exp(-jnp.abs(p)))
    lsn = jnp.minimum(-n, 0.0) - jnp.log1p(jnp.exp(-jnp.abs(n)))
    out_ref[0, 0] = -(jnp.sum(lsp) + jnp.sum(lsn)) / _B


def _finish(pos2d, neg2d):
    return pl.pallas_call(
        _finish_body,
        out_shape=jax.ShapeDtypeStruct((1, 1), jnp.float32),
        out_specs=pl.BlockSpec(memory_space=pltpu.SMEM),
    )(pos2d, neg2d)


def kernel(inputs, targets, negatives, emb_u, emb_v):
    idxu = inputs.astype(jnp.int32).reshape(-1)
    idxn = negatives.astype(jnp.int32).reshape(-1)
    idxt = targets.astype(jnp.int32)
    su = _phase_a()(idxu, emb_u)
    pos, neg = _phase_b()(idxn, idxt, su, emb_v)
    res = _finish(pos.reshape(128, 128), neg.reshape(128, 128))
    return res[0, 0]


# speedup vs baseline: 2.0998x; 1.0149x over previous
import functools

import jax
import jax.numpy as jnp
from jax import lax
from jax.experimental import pallas as pl
from jax.experimental.pallas import tpu as pltpu
from jax.experimental.pallas import tpu_sc as plsc

_V, _D = 1_000_000, 64
_B, _C, _K = 16384, 20, 20
_NC, _NS = 2, 16
_NW = _NC * _NS
_BPW = _B // _NW
_NB = 16
_NCHUNK = _BPW // _NB
_ROWS = _NB * _C
_NSTREAM = 4
_SROWS = _ROWS // _NSTREAM
_L = 16


def _phase_a_body(idxu_hbm, emb_u_hbm, su_hbm,
                  idx_u, rows_u, rows_u2, su_buf, sem):
    wid = lax.axis_index("s") * _NC + lax.axis_index("c")
    base = wid * _BPW
    pltpu.sync_copy(idxu_hbm.at[pl.ds(base * _C, _BPW * _C)], idx_u)

    def chunk_body(pi, _):
        cps = []
        for half, rows in ((0, rows_u), (1, rows_u2)):
            off = (2 * pi + half) * _ROWS
            cps.append([pltpu.async_copy(
                emb_u_hbm.at[idx_u.at[pl.ds(off + s * _SROWS, _SROWS)]],
                rows.at[pl.ds(s * _SROWS, _SROWS), :], sem)
                for s in range(_NSTREAM)])

        for half, rows in ((0, rows_u), (1, rows_u2)):
            ci = 2 * pi + half
            for cp in cps[half]:
                cp.wait()

            def b_body(j, _, rows=rows, ci=ci):
                rb = j * _C
                su = [jnp.zeros((_L,), jnp.float32) for _ in range(_D // _L)]
                for c in range(_C):
                    for blk in range(_D // _L):
                        su[blk] = su[blk] + rows[rb + c, pl.ds(blk * _L, _L)]
                for blk in range(_D // _L):
                    su_buf[ci * _NB + j, pl.ds(blk * _L, _L)] = su[blk]
                return 0

            lax.fori_loop(0, _NB, b_body, 0)
        return 0

    lax.fori_loop(0, _NCHUNK // 2, chunk_body, 0)
    pltpu.sync_copy(su_buf, su_hbm.at[pl.ds(base, _BPW), :])


def _phase_b_body(idxn_hbm, idxt_hbm, su_hbm, emb_v_hbm,
                  pos_hbm, neg_hbm,
                  idx_n, idx_t, rows_n, rows_n2, rows_t, rows_t2, su_buf,
                  pos_buf, neg_buf, sem):
    wid = lax.axis_index("s") * _NC + lax.axis_index("c")
    base = wid * _BPW
    pltpu.sync_copy(idxn_hbm.at[pl.ds(base * _K, _BPW * _K)], idx_n)
    pltpu.sync_copy(idxt_hbm.at[pl.ds(base, _BPW)], idx_t)
    pltpu.sync_copy(su_hbm.at[pl.ds(base, _BPW), :], su_buf)

    lanes = lax.iota(jnp.int32, _L)

    def _lane_sum(x):
        for s in (8, 4, 2, 1):
            x = x + x.at[(lanes + s) % _L].get(mode="promise_in_bounds")
        return x

    def chunk_body(pi, _):
        cps = []
        for half, rows, rt in ((0, rows_n, rows_t), (1, rows_n2, rows_t2)):
            ci = 2 * pi + half
            off = ci * _ROWS
            c_list = [pltpu.async_copy(
                emb_v_hbm.at[idx_n.at[pl.ds(off + s * _SROWS, _SROWS)]],
                rows.at[pl.ds(s * _SROWS, _SROWS), :], sem)
                for s in range(_NSTREAM)]
            c_list.append(pltpu.async_copy(
                emb_v_hbm.at[idx_t.at[pl.ds(ci * _NB, _NB)]], rt, sem))
            cps.append(c_list)

        for half, rows, rt in ((0, rows_n, rows_t), (1, rows_n2, rows_t2)):
            ci = 2 * pi + half
            for cp in cps[half]:
                cp.wait()

            def b_body(j, carry, rows=rows, rt=rt, ci=ci):
                acc_p, acc_n = carry
                rb = j * _C
                sn = [jnp.zeros((_L,), jnp.float32) for _ in range(_D // _L)]
                for c in range(_C):
                    for blk in range(_D // _L):
                        sn[blk] = sn[blk] + rows[rb + c, pl.ds(blk * _L, _L)]
                pp = jnp.zeros((_L,), jnp.float32)
                nn = jnp.zeros((_L,), jnp.float32)
                for blk in range(_D // _L):
                    su_blk = su_buf[ci * _NB + j, pl.ds(blk * _L, _L)]
                    t_blk = rt[j, pl.ds(blk * _L, _L)]
                    pp = pp + t_blk * su_blk
                    nn = nn + sn[blk] * su_blk
                m = lanes == j
                acc_p = jnp.where(m, _lane_sum(pp) * (1.0 / _C), acc_p)
                acc_n = jnp.where(m, _lane_sum(nn) * (1.0 / _C), acc_n)
                return acc_p, acc_n

            acc_p, acc_n = lax.fori_loop(
                0, _NB, b_body,
                (jnp.zeros((_L,), jnp.float32),
                 jnp.zeros((_L,), jnp.float32)))
            pos_buf[pl.ds(ci * _NB, _NB)] = acc_p
            neg_buf[pl.ds(ci * _NB, _NB)] = acc_n
        return 0

    lax.fori_loop(0, _NCHUNK // 2, chunk_body, 0)
    pltpu.sync_copy(pos_buf, pos_hbm.at[pl.ds(base, _BPW)])
    pltpu.sync_copy(neg_buf, neg_hbm.at[pl.ds(base, _BPW)])


_MESH = dict(core_axis_name="c", subcore_axis_name="s")


@functools.lru_cache(maxsize=None)
def _phase_a():
    return functools.partial(
        pl.kernel,
        mesh=plsc.VectorSubcoreMesh(**_MESH),
        compiler_params=pltpu.CompilerParams(use_tc_tiling_on_sc=False),
        out_type=jax.ShapeDtypeStruct((_B, _D), jnp.float32),
        scratch_types=[
            pltpu.VMEM((_BPW * _C,), jnp.int32),
            pltpu.VMEM((_ROWS, _D), jnp.float32),
            pltpu.VMEM((_ROWS, _D), jnp.float32),
            pltpu.VMEM((_BPW, _D), jnp.float32),
            pltpu.SemaphoreType.DMA,
        ],
    )(_phase_a_body)


@functools.lru_cache(maxsize=None)
def _phase_b():
    return functools.partial(
        pl.kernel,
        mesh=plsc.VectorSubcoreMesh(**_MESH),
        compiler_params=pltpu.CompilerParams(use_tc_tiling_on_sc=False),
        out_type=[jax.ShapeDtypeStruct((_B,), jnp.float32),
                  jax.ShapeDtypeStruct((_B,), jnp.float32)],
        scratch_types=[
            pltpu.VMEM((_BPW * _K,), jnp.int32),
            pltpu.VMEM((_BPW,), jnp.int32),
            pltpu.VMEM((_ROWS, _D), jnp.float32),
            pltpu.VMEM((_ROWS, _D), jnp.float32),
            pltpu.VMEM((_NB, _D), jnp.float32),
            pltpu.VMEM((_NB, _D), jnp.float32),
            pltpu.VMEM((_BPW, _D), jnp.float32),
            pltpu.VMEM((_BPW,), jnp.float32),
            pltpu.VMEM((_BPW,), jnp.float32),
            pltpu.SemaphoreType.DMA,
        ],
    )(_phase_b_body)


def _finish_body(pos_ref, neg_ref, out_ref):
    p = pos_ref[...]
    n = neg_ref[...]
    lsp = jnp.minimum(p, 0.0) - jnp.log1p(jnp.exp(-jnp.abs(p)))
    lsn = jnp.minimum(-n, 0.0) - jnp.log1p(jnp.exp(-jnp.abs(n)))
    out_ref[0, 0] = -(jnp.sum(lsp) + jnp.sum(lsn)) / _B


def _finish(pos2d, neg2d):
    return pl.pallas_call(
        _finish_body,
        out_shape=jax.ShapeDtypeStruct((1, 1), jnp.float32),
        out_specs=pl.BlockSpec(memory_space=pltpu.SMEM),
    )(pos2d, neg2d)


def kernel(inputs, targets, negatives, emb_u, emb_v):
    idxu = inputs.astype(jnp.int32).reshape(-1)
    idxn = negatives.astype(jnp.int32).reshape(-1)
    idxt = targets.astype(jnp.int32)
    su = _phase_a()(idxu, emb_u)
    pos, neg = _phase_b()(idxn, idxt, su, emb_v)
    res = _finish(pos.reshape(128, 128), neg.reshape(128, 128))
    return res[0, 0]
